# Initial kernel scaffold; baseline (speedup 1.0000x reference)
import jax
import jax.numpy as jnp
from jax.experimental import pallas as pl


def kernel(t, dep, te, de, W, b, gamma, beta):
    raise NotImplementedError("write your pallas kernel here")


# speedup vs baseline: 8.1764x; 8.1764x over previous
import functools

import jax
import jax.numpy as jnp
from jax import lax
from jax.experimental import pallas as pl
from jax.experimental.pallas import tpu as pltpu
from jax.experimental.pallas import tpu_sc as plsc

D = 64
NTYPE_PAD = 64
VOC = NTYPE_PAD * 64

NC = 2
NS = 16
NW = NC * NS

SUB = 128
CHB = 8


def _table_body(te_ref, de_ref, w_ref, b_ref, g_ref, bt_ref, out_ref):
    wt = w_ref[:, :D]
    wd = w_ref[:, D:]
    e = te_ref[...]
    p_row = lax.dot_general(e, wt, (((1,), (1,)), ((), ())),
                            preferred_element_type=jnp.float32)
    p_dep = lax.dot_general(de_ref[...], wd, (((1,), (1,)), ((), ())),
                            preferred_element_type=jnp.float32)
    y = p_row[:, None, :] + p_dep[None, :, :] + b_ref[...][None, :, :]
    mu = jnp.mean(y, axis=-1, keepdims=True)
    var = jnp.mean((y - mu) ** 2, axis=-1, keepdims=True)
    out_ref[...] = ((y - mu) * lax.rsqrt(var + 1e-5) * g_ref[...][None, :, :]
                    + bt_ref[...][None, :, :])


def _build_table(te_pad, de, w, b, gamma, beta):
    res = pl.pallas_call(
        _table_body,
        grid=(NTYPE_PAD // 8,),
        in_specs=[
            pl.BlockSpec((8, D), lambda i: (i, 0)),
            pl.BlockSpec((64, D), lambda i: (0, 0)),
            pl.BlockSpec((D, 2 * D), lambda i: (0, 0)),
            pl.BlockSpec((1, D), lambda i: (0, 0)),
            pl.BlockSpec((1, D), lambda i: (0, 0)),
            pl.BlockSpec((1, D), lambda i: (0, 0)),
        ],
        out_specs=pl.BlockSpec((8, 64, D), lambda i: (i, 0, 0)),
        out_shape=jax.ShapeDtypeStruct((NTYPE_PAD, 64, D), jnp.float32),
    )(te_pad, de, w, b, gamma, beta)
    return res.reshape(VOC, D)


def _gather_body(nblocks_w, t_hbm, dep_hbm, tab_hbm, out_hbm,
                 t_v, dep_v, idx_v, rows_v, sem):
    wid = lax.axis_index("s") * NC + lax.axis_index("c")
    nchunks = nblocks_w // CHB

    def chunk(c, _):
        blk0 = wid * nblocks_w + c * CHB
        pltpu.sync_copy(t_hbm.at[pl.ds(blk0, CHB)], t_v)
        pltpu.sync_copy(dep_hbm.at[pl.ds(blk0, CHB)], dep_v)

        def cstep(s, _):
            r = s // 8
            k = (s % 8) * 16
            tt = t_v[r, pl.ds(k, 16)]
            dd = dep_v[r, pl.ds(k, 16)]
            dd = jnp.minimum(jnp.maximum(dd, 0), 63)
            idx_v[r, pl.ds(k, 16)] = tt * 64 + dd
            return 0

        lax.fori_loop(0, CHB * 8, cstep, 0, unroll=True)

        def gstep(j, _):
            pltpu.async_copy(tab_hbm.at[idx_v.at[j]], rows_v, sem).wait()
            pltpu.sync_copy(rows_v, out_hbm.at[pl.ds((blk0 + j) * SUB, SUB)])
            return 0

        lax.fori_loop(0, CHB, gstep, 0)
        return 0

    lax.fori_loop(0, nchunks, chunk, 0)


def kernel(t, dep, te, de, W, b, gamma, beta):
    Bn, Ln = t.shape
    tot = Bn * Ln
    nblk = tot // SUB
    nblocks_w = nblk // NW

    te_pad = jnp.zeros((NTYPE_PAD, D), jnp.float32).at[: te.shape[0]].set(te)
    table = _build_table(te_pad, de,
                         W.astype(jnp.float32),
                         b.reshape(1, D), gamma.reshape(1, D),
                         beta.reshape(1, D))

    t2 = t.astype(jnp.int32).reshape(nblk, SUB)
    dep2 = dep.astype(jnp.int32).reshape(nblk, SUB)

    gather = pl.kernel(
        functools.partial(_gather_body, nblocks_w),
        out_type=jax.ShapeDtypeStruct((tot, D), jnp.float32),
        mesh=plsc.VectorSubcoreMesh(core_axis_name="c", subcore_axis_name="s"),
        scratch_types=[
            pltpu.VMEM((CHB, SUB), jnp.int32),
            pltpu.VMEM((CHB, SUB), jnp.int32),
            pltpu.VMEM((CHB, SUB), jnp.int32),
            pltpu.VMEM((SUB, D), jnp.float32),
            pltpu.SemaphoreType.DMA,
        ],
        compiler_params=pltpu.CompilerParams(use_tc_tiling_on_sc=False),
    )
    out = gather(t2, dep2, table)
    return out.reshape(Bn, Ln, D)


# speedup vs baseline: 9.5677x; 1.1701x over previous
import functools

import jax
import jax.numpy as jnp
from jax import lax
from jax.experimental import pallas as pl
from jax.experimental.pallas import tpu as pltpu
from jax.experimental.pallas import tpu_sc as plsc

D = 64
NTYPE_PAD = 64
VOC = NTYPE_PAD * 64

NC = 2
NS = 16
NW = NC * NS

SUB = 128
CHB = 16
NB = 4


def _table_body(te_ref, de_ref, w_ref, b_ref, g_ref, bt_ref, out_ref):
    wt = w_ref[:, :D]
    wd = w_ref[:, D:]
    e = te_ref[...]
    p_row = lax.dot_general(e, wt, (((1,), (1,)), ((), ())),
                            preferred_element_type=jnp.float32)
    p_dep = lax.dot_general(de_ref[...], wd, (((1,), (1,)), ((), ())),
                            preferred_element_type=jnp.float32)
    y = p_row[:, None, :] + p_dep[None, :, :] + b_ref[...][None, :, :]
    mu = jnp.mean(y, axis=-1, keepdims=True)
    var = jnp.mean((y - mu) ** 2, axis=-1, keepdims=True)
    out_ref[...] = ((y - mu) * lax.rsqrt(var + 1e-5) * g_ref[...][None, :, :]
                    + bt_ref[...][None, :, :])


def _build_table(te_pad, de, w, b, gamma, beta):
    res = pl.pallas_call(
        _table_body,
        grid=(NTYPE_PAD // 8,),
        in_specs=[
            pl.BlockSpec((8, D), lambda i: (i, 0)),
            pl.BlockSpec((64, D), lambda i: (0, 0)),
            pl.BlockSpec((D, 2 * D), lambda i: (0, 0)),
            pl.BlockSpec((1, D), lambda i: (0, 0)),
            pl.BlockSpec((1, D), lambda i: (0, 0)),
            pl.BlockSpec((1, D), lambda i: (0, 0)),
        ],
        out_specs=pl.BlockSpec((8, 64, D), lambda i: (i, 0, 0)),
        out_shape=jax.ShapeDtypeStruct((NTYPE_PAD, 64, D), jnp.float32),
    )(te_pad, de, w, b, gamma, beta)
    return res.reshape(VOC, D)


def _gather_body(nblocks_w, t_hbm, dep_hbm, tab_hbm, out_hbm,
                 t_v, dep_v, idx_v, rows_v, *sems):
    gsems, wsems = sems[:NB], sems[NB:]
    wid = lax.axis_index("s") * NC + lax.axis_index("c")
    nchunks = nblocks_w // CHB

    def chunk(c, _):
        blk0 = wid * nblocks_w + c * CHB
        pltpu.sync_copy(t_hbm.at[pl.ds(blk0, CHB)], t_v)
        pltpu.sync_copy(dep_hbm.at[pl.ds(blk0, CHB)], dep_v)

        def cstep(s, _):
            r = s // 8
            k = (s % 8) * 16
            tt = t_v[r, pl.ds(k, 16)]
            dd = dep_v[r, pl.ds(k, 16)]
            dd = jnp.minimum(jnp.maximum(dd, 0), 63)
            idx_v[r, pl.ds(k, 16)] = tt * 64 + dd
            return 0

        lax.fori_loop(0, CHB * 8, cstep, 0)

        def start_g(j, b):
            return pltpu.async_copy(tab_hbm.at[idx_v.at[j]],
                                    rows_v.at[b], gsems[b])

        def start_w(j, b):
            return pltpu.async_copy(rows_v.at[b],
                                    out_hbm.at[pl.ds((blk0 + j) * SUB, SUB)],
                                    wsems[b])

        gds = [start_g(b, b) for b in range(NB)]
        for j in range(CHB):
            b = j % NB
            gds[b].wait()
            wd = start_w(j, b)
            if j + NB < CHB:
                wd.wait()
                gds[b] = start_g(j + NB, b)
            else:
                wd.wait()
        return 0

    lax.fori_loop(0, nchunks, chunk, 0)


def kernel(t, dep, te, de, W, b, gamma, beta):
    Bn, Ln = t.shape
    tot = Bn * Ln
    nblk = tot // SUB
    nblocks_w = nblk // NW

    te_pad = jnp.zeros((NTYPE_PAD, D), jnp.float32).at[: te.shape[0]].set(te)
    table = _build_table(te_pad, de,
                         W.astype(jnp.float32),
                         b.reshape(1, D), gamma.reshape(1, D),
                         beta.reshape(1, D))

    t2 = t.astype(jnp.int32).reshape(nblk, SUB)
    dep2 = dep.astype(jnp.int32).reshape(nblk, SUB)

    gather = pl.kernel(
        functools.partial(_gather_body, nblocks_w),
        out_type=jax.ShapeDtypeStruct((tot, D), jnp.float32),
        mesh=plsc.VectorSubcoreMesh(core_axis_name="c", subcore_axis_name="s"),
        scratch_types=[
            pltpu.VMEM((CHB, SUB), jnp.int32),
            pltpu.VMEM((CHB, SUB), jnp.int32),
            pltpu.VMEM((CHB, SUB), jnp.int32),
            pltpu.VMEM((NB, SUB, D), jnp.float32),
        ] + [pltpu.SemaphoreType.DMA] * (2 * NB),
        compiler_params=pltpu.CompilerParams(use_tc_tiling_on_sc=False),
    )
    out = gather(t2, dep2, table)
    return out.reshape(Bn, Ln, D)


# speedup vs baseline: 11.3078x; 1.1819x over previous
import functools

import jax
import jax.numpy as jnp
from jax import lax
from jax.experimental import pallas as pl
from jax.experimental.pallas import tpu as pltpu
from jax.experimental.pallas import tpu_sc as plsc

D = 64
NTYPE_PAD = 64
VOC = NTYPE_PAD * 64

NC = 2
NS = 16
NW = NC * NS

SUB = 128
CHB = 16
NB = 4


def _table_body(te_ref, de_ref, w_ref, b_ref, g_ref, bt_ref, out_ref):
    wt = w_ref[:, :D]
    wd = w_ref[:, D:]
    e = te_ref[...]
    p_row = lax.dot_general(e, wt, (((1,), (1,)), ((), ())),
                            preferred_element_type=jnp.float32)
    p_dep = lax.dot_general(de_ref[...], wd, (((1,), (1,)), ((), ())),
                            preferred_element_type=jnp.float32)
    y = p_row[:, None, :] + p_dep[None, :, :] + b_ref[...][None, :, :]
    mu = jnp.mean(y, axis=-1, keepdims=True)
    var = jnp.mean((y - mu) ** 2, axis=-1, keepdims=True)
    out_ref[...] = ((y - mu) * lax.rsqrt(var + 1e-5) * g_ref[...][None, :, :]
                    + bt_ref[...][None, :, :])


def _build_table(te_pad, de, w, b, gamma, beta):
    res = pl.pallas_call(
        _table_body,
        grid=(NTYPE_PAD // 8,),
        in_specs=[
            pl.BlockSpec((8, D), lambda i: (i, 0)),
            pl.BlockSpec((64, D), lambda i: (0, 0)),
            pl.BlockSpec((D, 2 * D), lambda i: (0, 0)),
            pl.BlockSpec((1, D), lambda i: (0, 0)),
            pl.BlockSpec((1, D), lambda i: (0, 0)),
            pl.BlockSpec((1, D), lambda i: (0, 0)),
        ],
        out_specs=pl.BlockSpec((8, 64, D), lambda i: (i, 0, 0)),
        out_shape=jax.ShapeDtypeStruct((NTYPE_PAD, 64, D), jnp.float32),
    )(te_pad, de, w, b, gamma, beta)
    return res.reshape(VOC, D)


def _gather_body(nblocks_w, t_hbm, dep_hbm, tab_hbm, out_hbm,
                 t_v, dep_v, idx_v, rows_v, tab_sh, *sems):
    gsems, wsems = sems[:NB], sems[NB:]
    sid = lax.axis_index("s")
    wid = sid * NC + lax.axis_index("c")
    nchunks = nblocks_w // CHB

    @pl.when(sid == 0)
    def _():
        pltpu.sync_copy(tab_hbm, tab_sh)

    plsc.subcore_barrier()

    def chunk(c, _):
        blk0 = wid * nblocks_w + c * CHB
        pltpu.sync_copy(t_hbm.at[pl.ds(blk0, CHB)], t_v)
        pltpu.sync_copy(dep_hbm.at[pl.ds(blk0, CHB)], dep_v)

        def cstep(s, _):
            r = s // 8
            k = (s % 8) * 16
            tt = t_v[r, pl.ds(k, 16)]
            dd = dep_v[r, pl.ds(k, 16)]
            dd = jnp.minimum(jnp.maximum(dd, 0), 63)
            idx_v[r, pl.ds(k, 16)] = tt * 64 + dd
            return 0

        lax.fori_loop(0, CHB * 8, cstep, 0)

        def start_g(j, b):
            return pltpu.async_copy(tab_sh.at[idx_v.at[j]],
                                    rows_v.at[b], gsems[b])

        def start_w(j, b):
            return pltpu.async_copy(rows_v.at[b],
                                    out_hbm.at[pl.ds((blk0 + j) * SUB, SUB)],
                                    wsems[b])

        gds = [start_g(b, b) for b in range(NB)]
        wds = [None] * NB
        for j in range(CHB):
            b = j % NB
            gds[b].wait()
            wds[b] = start_w(j, b)
            if j + NB < CHB:
                wds[b].wait()
                gds[b] = start_g(j + NB, b)
        for b in range(NB):
            wds[b].wait()
        return 0

    lax.fori_loop(0, nchunks, chunk, 0)


def kernel(t, dep, te, de, W, b, gamma, beta):
    Bn, Ln = t.shape
    tot = Bn * Ln
    nblk = tot // SUB
    nblocks_w = nblk // NW

    te_pad = jnp.zeros((NTYPE_PAD, D), jnp.float32).at[: te.shape[0]].set(te)
    table = _build_table(te_pad, de,
                         W.astype(jnp.float32),
                         b.reshape(1, D), gamma.reshape(1, D),
                         beta.reshape(1, D))

    t2 = t.astype(jnp.int32).reshape(nblk, SUB)
    dep2 = dep.astype(jnp.int32).reshape(nblk, SUB)

    gather = pl.kernel(
        functools.partial(_gather_body, nblocks_w),
        out_type=jax.ShapeDtypeStruct((tot, D), jnp.float32),
        mesh=plsc.VectorSubcoreMesh(core_axis_name="c", subcore_axis_name="s"),
        scratch_types=[
            pltpu.VMEM((CHB, SUB), jnp.int32),
            pltpu.VMEM((CHB, SUB), jnp.int32),
            pltpu.VMEM((CHB, SUB), jnp.int32),
            pltpu.VMEM((NB, SUB, D), jnp.float32),
            pltpu.VMEM_SHARED((VOC, D), jnp.float32),
        ] + [pltpu.SemaphoreType.DMA] * (2 * NB),
        compiler_params=pltpu.CompilerParams(use_tc_tiling_on_sc=False),
    )
    out = gather(t2, dep2, table)
    return out.reshape(Bn, Ln, D)
